# Initial kernel scaffold; baseline (speedup 1.0000x reference)
import jax
import jax.numpy as jnp
from jax.experimental import pallas as pl


def kernel(x, W, b, k, nn_index):
    raise NotImplementedError("write your pallas kernel here")


# speedup vs baseline: 6.1729x; 6.1729x over previous
import functools

import jax
import jax.numpy as jnp
from jax import lax
from jax.experimental import pallas as pl
from jax.experimental.pallas import tpu as pltpu
from jax.experimental.pallas import tpu_sc as plsc

N = 10000
E = 320000
D = 128
H = D // 2
NS = 16
EPS = E // NS
CH = 80
NCHUNK = EPS // CH
RPT = 624
REM = N - NS * RPT
ZR = 208


def _prep_body(x_ref, w_ref, b_ref, a0_ref, a1_ref, b0_ref, b1_ref):
    w1 = w_ref[0:D, :]
    w2 = w_ref[D : 2 * D, :]
    xb = x_ref[...]
    a = jnp.dot(xb, w1 - w2, preferred_element_type=jnp.float32) + b_ref[...]
    bb = jnp.dot(xb, w2, preferred_element_type=jnp.float32)
    a0_ref[...] = a[:, 0:H]
    a1_ref[...] = a[:, H:D]
    b0_ref[...] = bb[:, 0:H]
    b1_ref[...] = bb[:, H:D]


def _prep(x, w, b2d):
    bm = 1000
    half = jax.ShapeDtypeStruct((N, H), jnp.float32)
    return pl.pallas_call(
        _prep_body,
        grid=(N // bm,),
        in_specs=[
            pl.BlockSpec((bm, D), lambda i: (i, 0)),
            pl.BlockSpec((2 * D, D), lambda i: (0, 0)),
            pl.BlockSpec((1, D), lambda i: (0, 0)),
        ],
        out_specs=[pl.BlockSpec((bm, H), lambda i: (i, 0))] * 4,
        out_shape=[half] * 4,
    )(x, w, b2d)


_MESH = plsc.VectorSubcoreMesh(core_axis_name="c", subcore_axis_name="s")


@functools.partial(
    pl.kernel,
    mesh=_MESH,
    compiler_params=pltpu.CompilerParams(use_tc_tiling_on_sc=False),
    out_type=[
        jax.ShapeDtypeStruct((2, N, H), jnp.float32),
        jax.ShapeDtypeStruct((2, N, 16), jnp.float32),
    ],
    scratch_types=[
        pltpu.VMEM((NCHUNK, CH), jnp.int32),
        pltpu.VMEM((NCHUNK, CH), jnp.int32),
        pltpu.VMEM((CH, H), jnp.float32),
        pltpu.VMEM((CH, H), jnp.float32),
        pltpu.VMEM((CH, H), jnp.float32),
        pltpu.VMEM((CH, 16), jnp.float32),
        pltpu.VMEM((ZR, H), jnp.float32),
        pltpu.VMEM((RPT, 16), jnp.float32),
        pltpu.VMEM_SHARED((N, H), jnp.float32),
        pltpu.VMEM_SHARED((N, 16), jnp.float32),
        pltpu.SemaphoreType.DMA,
        pltpu.SemaphoreType.DMA,
    ],
)
def _sc_main(
    tgt_hbm, src_hbm, a0_hbm, a1_hbm, b0_hbm, b1_hbm,
    out_hbm, deg_hbm,
    tgt_v, src_v, buf_a, buf_b, buf_m, ones_v, zrow_v, zdeg_v,
    acc_sh, deg_sh, sem_a, sem_b,
):
    cid = lax.axis_index("c")
    sid = lax.axis_index("s")

    pltpu.sync_copy(tgt_hbm.at[sid], tgt_v)
    pltpu.sync_copy(src_hbm.at[sid], src_v)

    zero16 = jnp.zeros((16,), jnp.float32)
    one16 = jnp.ones((16,), jnp.float32)

    def _fill_zrow(i, carry):
        for g in range(H // 16):
            zrow_v[i, pl.ds(g * 16, 16)] = zero16
        return carry

    lax.fori_loop(0, ZR, _fill_zrow, 0)

    def _fill_zdeg(i, carry):
        zdeg_v[i, :] = zero16
        return carry

    lax.fori_loop(0, RPT, _fill_zdeg, 0)

    def _fill_ones(i, carry):
        ones_v[i, :] = one16
        return carry

    lax.fori_loop(0, CH, _fill_ones, 0)

    for r in range(RPT // ZR):
        pltpu.sync_copy(zrow_v, acc_sh.at[pl.ds(sid * RPT + r * ZR, ZR)])
    pltpu.sync_copy(zdeg_v, deg_sh.at[pl.ds(sid * RPT, RPT)])

    @pl.when(sid == 15)
    def _zero_tail():
        pltpu.sync_copy(zrow_v.at[pl.ds(0, REM)],
                        acc_sh.at[pl.ds(NS * RPT, REM)])
        pltpu.sync_copy(zdeg_v.at[pl.ds(0, REM)],
                        deg_sh.at[pl.ds(NS * RPT, REM)])

    plsc.subcore_barrier()

    def _chunk(ci, carry):
        ti = tgt_v.at[ci]
        si = src_v.at[ci]

        @pl.when(cid == 0)
        def _gather0():
            cp_a = pltpu.async_copy(a0_hbm.at[ti], buf_a, sem_a)
            cp_b = pltpu.async_copy(b0_hbm.at[si], buf_b, sem_b)
            cp_a.wait()
            cp_b.wait()

        @pl.when(cid == 1)
        def _gather1():
            cp_a = pltpu.async_copy(a1_hbm.at[ti], buf_a, sem_a)
            cp_b = pltpu.async_copy(b1_hbm.at[si], buf_b, sem_b)
            cp_a.wait()
            cp_b.wait()

        def _edge(e, c2):
            for g in range(H // 16):
                av = buf_a[e, pl.ds(g * 16, 16)]
                bv = buf_b[e, pl.ds(g * 16, 16)]
                buf_m[e, pl.ds(g * 16, 16)] = jnp.maximum(av + bv, 0.0)
            return c2

        lax.fori_loop(0, CH, _edge, 0)
        pltpu.sync_copy(buf_m, acc_sh.at[ti], add=True)
        pltpu.sync_copy(ones_v, deg_sh.at[ti], add=True)
        return carry

    lax.fori_loop(0, NCHUNK, _chunk, 0)
    plsc.subcore_barrier()

    pltpu.sync_copy(acc_sh.at[pl.ds(sid * RPT, RPT)],
                    out_hbm.at[cid, pl.ds(sid * RPT, RPT)])
    pltpu.sync_copy(deg_sh.at[pl.ds(sid * RPT, RPT)],
                    deg_hbm.at[cid, pl.ds(sid * RPT, RPT)])

    @pl.when(sid == 15)
    def _write_tail():
        pltpu.sync_copy(acc_sh.at[pl.ds(NS * RPT, REM)],
                        out_hbm.at[cid, pl.ds(NS * RPT, REM)])
        pltpu.sync_copy(deg_sh.at[pl.ds(NS * RPT, REM)],
                        deg_hbm.at[cid, pl.ds(NS * RPT, REM)])


def _fin_body(acc_ref, deg_ref, o_ref):
    d0 = deg_ref[0, :, 0:1] + 1e-8
    d1 = deg_ref[1, :, 0:1] + 1e-8
    o_ref[:, 0:H] = acc_ref[0, :, :] / d0
    o_ref[:, H:D] = acc_ref[1, :, :] / d1


def _finalize(acc, deg):
    bm = 1000
    return pl.pallas_call(
        _fin_body,
        grid=(N // bm,),
        in_specs=[
            pl.BlockSpec((2, bm, H), lambda i: (0, i, 0)),
            pl.BlockSpec((2, bm, 16), lambda i: (0, i, 0)),
        ],
        out_specs=pl.BlockSpec((bm, D), lambda i: (i, 0)),
        out_shape=jax.ShapeDtypeStruct((N, D), jnp.float32),
    )(acc, deg)


def kernel(x, W, b, k, nn_index):
    a0, a1, b0, b1 = _prep(x, W, b.reshape(1, D))
    src = nn_index[0].astype(jnp.int32).reshape(NS, NCHUNK, CH)
    tgt = nn_index[1].astype(jnp.int32).reshape(NS, NCHUNK, CH)
    acc, deg = _sc_main(tgt, src, a0, a1, b0, b1)
    return _finalize(acc, deg)
